# baseline (speedup 1.0000x reference)
import functools

import jax
import jax.numpy as jnp
from jax import lax
from jax.experimental import pallas as pl
from jax.experimental.pallas import tpu as pltpu
from jax.experimental.pallas import tpu_sc as plsc

DIM = 64
NC = 2
NS = 16
NW = NC * NS
K = 128
M = 10
G = 5


@functools.cache
def _make(total_rows: int):
  cpw = total_rows // (NW * K)
  assert cpw % M == 0 and cpw >= 2 * M
  mesh = plsc.VectorSubcoreMesh(core_axis_name="c", subcore_axis_name="s")

  @functools.partial(
      pl.kernel,
      out_type=jax.ShapeDtypeStruct((total_rows, DIM), jnp.float32),
      mesh=mesh,
      scratch_types=[
          pltpu.VMEM((cpw, K), jnp.int32),
          pltpu.VMEM((M, K, DIM), jnp.float32),
          pltpu.SemaphoreType.DMA((M,)),
          pltpu.SemaphoreType.DMA((M,)),
      ],
      compiler_params=pltpu.CompilerParams(use_tc_tiling_on_sc=False),
  )
  def gather_kernel(ids_hbm, table_hbm, out_hbm, idx_v, bufs, gsem, osem):
    wid = lax.axis_index("s") * NC + lax.axis_index("c")
    row0 = wid * cpw
    pltpu.sync_copy(ids_hbm.at[pl.ds(row0, cpw)], idx_v)

    def start_gather(j, s):
      pltpu.async_copy(table_hbm.at[idx_v.at[j]], bufs.at[s], gsem.at[s])

    def wait_gather(s):
      pltpu.make_async_copy(
          table_hbm.at[idx_v.at[0]], bufs.at[s], gsem.at[s]).wait()

    def start_out(j, s):
      pltpu.async_copy(
          bufs.at[s], out_hbm.at[pl.ds((row0 + j) * K, K)], osem.at[s])

    def wait_out(s):
      pltpu.make_async_copy(
          bufs.at[s], out_hbm.at[pl.ds(0, K)], osem.at[s]).wait()

    for j in range(G):
      start_gather(j, j)
    for j in range(M - G):
      wait_gather(j)
      start_out(j, j)
      start_gather(j + G, j + G)

    base = M - G

    def body(t, carry):
      for b in range(M):
        j = base + t * M + b
        s = (base + b) % M
        wait_gather(s)
        start_out(j, s)
        wait_out(b)
        start_gather(j + G, b)
      return carry

    lax.fori_loop(0, (cpw - M) // M, body, 0)

    for i in range(G):
      j = cpw - G + i
      s = j % M
      wait_gather(s)
      start_out(j, s)
    for s in range(M):
      wait_out(s)

  return gather_kernel


def kernel(token_ids, embeddings):
  batch, hist = token_ids.shape
  ids = token_ids.reshape(-1, K).astype(jnp.int32)
  out = _make(batch * hist)(ids, embeddings)
  return out.reshape(batch, hist, DIM)


# speedup vs baseline: 1.0030x; 1.0030x over previous
import functools

import jax
import jax.numpy as jnp
from jax import lax
from jax.experimental import pallas as pl
from jax.experimental.pallas import tpu as pltpu
from jax.experimental.pallas import tpu_sc as plsc

DIM = 64
NC = 2
NS = 16
NW = NC * NS
K = 128
M = 10
G = 5


@functools.cache
def _make(total_rows: int):
  cpw = total_rows // (NW * K)
  assert cpw % M == 0 and cpw >= 2 * M
  mesh = plsc.VectorSubcoreMesh(core_axis_name="c", subcore_axis_name="s")

  @functools.partial(
      pl.kernel,
      out_type=jax.ShapeDtypeStruct((total_rows, DIM), jnp.float32),
      mesh=mesh,
      scratch_types=[
          pltpu.VMEM((cpw, K), jnp.int32),
          pltpu.VMEM((M, K, DIM), jnp.float32),
          pltpu.SemaphoreType.DMA((M,)),
          pltpu.SemaphoreType.DMA((M,)),
      ],
      compiler_params=pltpu.CompilerParams(use_tc_tiling_on_sc=False),
  )
  def gather_kernel(ids_hbm, table_hbm, out_hbm, idx_v, bufs, gsem, osem):
    wid = lax.axis_index("s") * NC + lax.axis_index("c")
    row0 = wid * cpw
    pltpu.sync_copy(ids_hbm.at[pl.ds(row0, cpw)], idx_v)

    def start_gather(j, s):
      pltpu.async_copy(table_hbm.at[idx_v.at[j]], bufs.at[s], gsem.at[s])

    def wait_gather(s):
      pltpu.make_async_copy(
          table_hbm.at[idx_v.at[0]], bufs.at[s], gsem.at[s]).wait()

    def start_out(j, s):
      pltpu.async_copy(
          bufs.at[s], out_hbm.at[pl.ds((row0 + j) * K, K)], osem.at[s])

    def wait_out(s):
      pltpu.make_async_copy(
          bufs.at[s], out_hbm.at[pl.ds(0, K)], osem.at[s]).wait()

    for j in range(G):
      start_gather(j, j)
    for j in range(M - G):
      wait_gather(j)
      start_out(j, j)
      start_gather(j + G, j + G)

    base = M - G

    def body(t, carry):
      for b in range(M):
        j = base + t * M + b
        s = (base + b) % M
        wait_gather(s)
        start_out(j, s)
        wait_out(b)
        start_gather(j + G, b)
      return carry

    lax.fori_loop(0, (cpw - M) // M, body, 0)

    for i in range(G):
      j = cpw - G + i
      s = j % M
      wait_gather(s)
      start_out(j, s)
    for s in range(M):
      wait_out(s)

  return gather_kernel


def kernel(token_ids, embeddings):
  batch, hist = token_ids.shape
  ids = token_ids.reshape(-1, K).astype(jnp.int32)
  nrows = embeddings.shape[0]
  tlin = jax.lax.optimization_barrier(
      embeddings.reshape(nrows // 2, 2 * DIM)).reshape(nrows, DIM)
  out = _make(batch * hist)(ids, tlin)
  return out.reshape(batch, hist, DIM)
